# baseline (speedup 1.0000x reference)
import functools

import jax
import jax.numpy as jnp
from jax import lax
from jax.experimental import pallas as pl
from jax.experimental.pallas import tpu as pltpu
from jax.experimental.pallas import tpu_sc as plsc

N = 10000
E = 320000
D = 128

NC = 2
NS = 16
NW = NC * NS

K = 128
NB = 80
NBT = NW * NB
E_PAD = NBT * K
NB0 = 56
NB1 = 104
NBMX = max(NB0, NB1)
RSTRIDE = 632
RSPAN = 640

_mesh = plsc.VectorSubcoreMesh(core_axis_name="c", subcore_axis_name="s")
_sc_params = pltpu.CompilerParams(needs_layout_passes=False,
                                  use_tc_tiling_on_sc=False)


@functools.partial(
    pl.kernel,
    out_type=jax.ShapeDtypeStruct((NW, 1, N), jnp.float32),
    mesh=_mesh,
    scratch_types=[
        pltpu.VMEM((NB, K), jnp.int32),
        pltpu.VMEM((NB, K), jnp.float32),
        pltpu.VMEM((N,), jnp.float32),
    ],
    compiler_params=_sc_params,
)
def _sc_deg(col_hbm, ew_hbm, degp_hbm, col_v, ew_v, deg_v):
    cid = lax.axis_index("c")
    sid = lax.axis_index("s")
    t = cid * NS + sid
    pltpu.sync_copy(col_hbm.at[pl.ds(t * NB, NB)], col_v)
    pltpu.sync_copy(ew_hbm.at[pl.ds(t * NB, NB)], ew_v)

    def _zero(i, _):
        deg_v[pl.ds(i * 16, 16)] = jnp.zeros((16,), jnp.float32)
        return 0

    lax.fori_loop(0, N // 16, _zero, 0)

    def _acc(i, _):
        b = i // (K // 16)
        g = i % (K // 16)
        idx = col_v[b, pl.ds(g * 16, 16)]
        val = ew_v[b, pl.ds(g * 16, 16)]
        plsc.addupdate_scatter(deg_v, [idx], val)
        return 0

    lax.fori_loop(0, NB * (K // 16), _acc, 0)
    pltpu.sync_copy(deg_v, degp_hbm.at[t, 0])


DH = D // 2

def _sc_agg_body(row_hbm, col_hbm, ew_hbm, ya_hbm, yb_hbm, outa_hbm, outb_hbm,
                 row_v, col_v, ew_v, rows0_v, rows1_v, zbuf_v, acc_s,
                 sg0, sg1, ss0, ss1):
    cid = lax.axis_index("c")
    sid = lax.axis_index("s")
    nb = jnp.where(cid == 0, NB0, NB1)
    start = pl.multiple_of(
        jnp.where(cid == 0, sid * NB0, NS * NB0 + sid * NB1), 8)

    pltpu.sync_copy(row_hbm.at[pl.ds(start, NBMX)], row_v.at[pl.ds(0, NBMX)])
    pltpu.sync_copy(col_hbm.at[pl.ds(start, NBMX)], col_v)
    pltpu.sync_copy(ew_hbm.at[pl.ds(start, NBMX)], ew_v)
    for d in range(2):
        for j in range(K // 16):
            row_v[nb + d, pl.ds(j * 16, 16)] = jnp.zeros((16,), jnp.int32)

    def _zrow(i, _):
        for j in range(DH // 32):
            zbuf_v[i, pl.ds(j * 32, 32)] = jnp.zeros((32,), jnp.bfloat16)
        return 0

    lax.fori_loop(0, K, _zrow, 0)
    base = pl.multiple_of(jnp.minimum(sid * RSTRIDE, N - RSPAN), 8)

    def _scale(rows_v, b):
        def _sg(g, _):
            ew16 = ew_v[b, pl.ds(g * 16, 16)]
            for l in range(16):
                sv = jnp.full((16,), ew16[l], jnp.float32)
                sb = plsc.pack(sv, sv, format=plsc.PackFormat.INTERLEAVED)
                e = g * 16 + l
                for j in range(DH // 32):
                    sl = rows_v[e, pl.ds(j * 32, 32)]
                    rows_v[e, pl.ds(j * 32, 32)] = sl * sb
            return 0

        lax.fori_loop(0, K // 16, _sg, 0)

    for y_hbm, out_hbm in ((ya_hbm, outa_hbm), (yb_hbm, outb_hbm)):
        for c in range(RSPAN // K):
            pltpu.sync_copy(zbuf_v, acc_s.at[pl.ds(base + c * K, K)])
        plsc.subcore_barrier()

        pltpu.async_copy(y_hbm.at[row_v.at[0]], rows0_v, sg0)
        pltpu.async_copy(y_hbm.at[row_v.at[1]], rows1_v, sg1)

        def _batch2(i, _):
            b = i * 2
            pltpu.make_async_copy(y_hbm.at[row_v.at[b]], rows0_v, sg0).wait()
            _scale(rows0_v, b)
            cp0 = pltpu.async_copy(rows0_v, acc_s.at[col_v.at[b]], ss0,
                                   add=True)
            pltpu.make_async_copy(y_hbm.at[row_v.at[b + 1]], rows1_v,
                                  sg1).wait()
            _scale(rows1_v, b + 1)
            cp1 = pltpu.async_copy(rows1_v, acc_s.at[col_v.at[b + 1]], ss1,
                                   add=True)
            cp0.wait()
            pltpu.async_copy(y_hbm.at[row_v.at[b + 2]], rows0_v, sg0)
            cp1.wait()
            pltpu.async_copy(y_hbm.at[row_v.at[b + 3]], rows1_v, sg1)
            return 0

        lax.fori_loop(0, nb // 2, _batch2, 0)
        pltpu.make_async_copy(y_hbm.at[row_v.at[nb]], rows0_v, sg0).wait()
        pltpu.make_async_copy(y_hbm.at[row_v.at[nb + 1]], rows1_v, sg1).wait()
        plsc.subcore_barrier()

        for c in range(RSPAN // K):
            pltpu.sync_copy(acc_s.at[pl.ds(base + c * K, K)], rows0_v)
            pltpu.sync_copy(rows0_v, out_hbm.at[cid, pl.ds(base + c * K, K)])
        plsc.subcore_barrier()


_sc_agg = pl.kernel(
    _sc_agg_body,
    out_type=(jax.ShapeDtypeStruct((NC, N, DH), jnp.bfloat16),
              jax.ShapeDtypeStruct((NC, N, DH), jnp.bfloat16)),
    mesh=_mesh,
    scratch_types=[
        pltpu.VMEM((NBMX + 2, K), jnp.int32),
        pltpu.VMEM((NBMX, K), jnp.int32),
        pltpu.VMEM((NBMX, K), jnp.float32),
        pltpu.VMEM((K, DH), jnp.bfloat16),
        pltpu.VMEM((K, DH), jnp.bfloat16),
        pltpu.VMEM((K, DH), jnp.bfloat16),
        pltpu.VMEM_SHARED((N, DH), jnp.bfloat16),
        pltpu.SemaphoreType.DMA,
        pltpu.SemaphoreType.DMA,
        pltpu.SemaphoreType.DMA,
        pltpu.SemaphoreType.DMA,
    ],
    compiler_params=_sc_params,
)


R = 1000
GRID = N // R


def _tc_pre_body(degp_ref, x_ref, w1_ref,
                 y1a_ref, y1b_ref, xw1_ref, dinv_ref, dinv2_ref):
    deg = 1.0 + jnp.sum(degp_ref[...], axis=0)
    dinv = lax.rsqrt(deg)
    dinv2 = dinv * dinv
    xw = jnp.dot(x_ref[...], w1_ref[...], preferred_element_type=jnp.float32)
    y1 = (dinv * xw).astype(jnp.bfloat16)
    xw1_ref[...] = xw
    y1a_ref[...] = y1[:, :DH]
    y1b_ref[...] = y1[:, DH:]
    dinv_ref[...] = dinv
    dinv2_ref[...] = dinv2


_tc_pre = pl.pallas_call(
    _tc_pre_body,
    grid=(GRID,),
    in_specs=[
        pl.BlockSpec((NW, R, 1), lambda i: (0, i, 0)),
        pl.BlockSpec((R, D), lambda i: (i, 0)),
        pl.BlockSpec((D, D), lambda i: (0, 0)),
    ],
    out_specs=[
        pl.BlockSpec((R, DH), lambda i: (i, 0)),
        pl.BlockSpec((R, DH), lambda i: (i, 0)),
        pl.BlockSpec((R, D), lambda i: (i, 0)),
        pl.BlockSpec((R, 1), lambda i: (i, 0)),
        pl.BlockSpec((R, 1), lambda i: (i, 0)),
    ],
    out_shape=[
        jax.ShapeDtypeStruct((N, DH), jnp.bfloat16),
        jax.ShapeDtypeStruct((N, DH), jnp.bfloat16),
        jax.ShapeDtypeStruct((N, D), jnp.float32),
        jax.ShapeDtypeStruct((N, 1), jnp.float32),
        jax.ShapeDtypeStruct((N, 1), jnp.float32),
    ],
)


def _tc_mid_body(s1pa_ref, s1pb_ref, xw1_ref, dinv_ref, dinv2_ref, b1_ref,
                 w2_ref, xw2_ref, y2a_ref, y2b_ref):
    f32 = jnp.float32
    s1 = jnp.concatenate(
        [s1pa_ref[0].astype(f32) + s1pa_ref[1].astype(f32),
         s1pb_ref[0].astype(f32) + s1pb_ref[1].astype(f32)], axis=1)
    dinv = dinv_ref[...]
    pre = dinv * s1 + dinv2_ref[...] * xw1_ref[...] + b1_ref[...]
    h = jnp.where(pre > 0, pre, jnp.exp(pre) - 1.0)
    xw2 = jnp.dot(h, w2_ref[...], preferred_element_type=jnp.float32)
    y2 = (dinv * xw2).astype(jnp.bfloat16)
    xw2_ref[...] = xw2
    y2a_ref[...] = y2[:, :DH]
    y2b_ref[...] = y2[:, DH:]


_tc_mid = pl.pallas_call(
    _tc_mid_body,
    grid=(GRID,),
    in_specs=[
        pl.BlockSpec((NC, R, DH), lambda i: (0, i, 0)),
        pl.BlockSpec((NC, R, DH), lambda i: (0, i, 0)),
        pl.BlockSpec((R, D), lambda i: (i, 0)),
        pl.BlockSpec((R, 1), lambda i: (i, 0)),
        pl.BlockSpec((R, 1), lambda i: (i, 0)),
        pl.BlockSpec((1, D), lambda i: (0, 0)),
        pl.BlockSpec((D, D), lambda i: (0, 0)),
    ],
    out_specs=[
        pl.BlockSpec((R, D), lambda i: (i, 0)),
        pl.BlockSpec((R, DH), lambda i: (i, 0)),
        pl.BlockSpec((R, DH), lambda i: (i, 0)),
    ],
    out_shape=[
        jax.ShapeDtypeStruct((N, D), jnp.float32),
        jax.ShapeDtypeStruct((N, DH), jnp.bfloat16),
        jax.ShapeDtypeStruct((N, DH), jnp.bfloat16),
    ],
)


def _tc_post_body(s2pa_ref, s2pb_ref, xw2_ref, dinv_ref, dinv2_ref, b2_ref,
                  out_ref):
    f32 = jnp.float32
    s2 = jnp.concatenate(
        [s2pa_ref[0].astype(f32) + s2pa_ref[1].astype(f32),
         s2pb_ref[0].astype(f32) + s2pb_ref[1].astype(f32)], axis=1)
    out_ref[...] = (dinv_ref[...] * s2 + dinv2_ref[...] * xw2_ref[...]
                    + b2_ref[...])


_tc_post = pl.pallas_call(
    _tc_post_body,
    grid=(GRID,),
    in_specs=[
        pl.BlockSpec((NC, R, DH), lambda i: (0, i, 0)),
        pl.BlockSpec((NC, R, DH), lambda i: (0, i, 0)),
        pl.BlockSpec((R, D), lambda i: (i, 0)),
        pl.BlockSpec((R, 1), lambda i: (i, 0)),
        pl.BlockSpec((R, 1), lambda i: (i, 0)),
        pl.BlockSpec((1, D), lambda i: (0, 0)),
    ],
    out_specs=pl.BlockSpec((R, D), lambda i: (i, 0)),
    out_shape=jax.ShapeDtypeStruct((N, D), jnp.float32),
)


def kernel(x, edge_index, edge_attr, W1, b1, W2, b2):
    row = edge_index[0]
    col = edge_index[1]
    pad = E_PAD - E
    row_p = jnp.concatenate([row, jnp.zeros((pad,), jnp.int32)])
    col_p = jnp.concatenate([col, jnp.zeros((pad,), jnp.int32)])
    ew_p = jnp.concatenate([edge_attr, jnp.zeros((pad,), jnp.float32)])
    row3 = row_p.reshape(NBT, K)
    col3 = col_p.reshape(NBT, K)
    ew3 = ew_p.reshape(NBT, K)

    degp = _sc_deg(col3, ew3)
    y1a, y1b, xw1, dinv, dinv2 = _tc_pre(degp.reshape(NW, N, 1), x, W1)
    s1pa, s1pb = _sc_agg(row3, col3, ew3, y1a, y1b)
    xw2, y2a, y2b = _tc_mid(s1pa, s1pb, xw1, dinv, dinv2,
                            b1.reshape(1, D), W2)
    s2pa, s2pb = _sc_agg(row3, col3, ew3, y2a, y2b)
    out = _tc_post(s2pa, s2pb, xw2, dinv, dinv2, b2.reshape(1, D))
    return out


# speedup vs baseline: 1.0324x; 1.0324x over previous
import functools

import jax
import jax.numpy as jnp
from jax import lax
from jax.experimental import pallas as pl
from jax.experimental.pallas import tpu as pltpu
from jax.experimental.pallas import tpu_sc as plsc

N = 10000
E = 320000
D = 128

NC = 2
NS = 16
NW = NC * NS

K = 128
NB = 80
NBT = NW * NB
E_PAD = NBT * K
NB0 = 104
NB1 = 56
NBMX = max(NB0, NB1)
RSTRIDE = 632
RSPAN = 640

_mesh = plsc.VectorSubcoreMesh(core_axis_name="c", subcore_axis_name="s")
_sc_params = pltpu.CompilerParams(needs_layout_passes=False,
                                  use_tc_tiling_on_sc=False)


@functools.partial(
    pl.kernel,
    out_type=jax.ShapeDtypeStruct((NW, 1, N), jnp.float32),
    mesh=_mesh,
    scratch_types=[
        pltpu.VMEM((NB, K), jnp.int32),
        pltpu.VMEM((NB, K), jnp.float32),
        pltpu.VMEM((N,), jnp.float32),
    ],
    compiler_params=_sc_params,
)
def _sc_deg(col_hbm, ew_hbm, degp_hbm, col_v, ew_v, deg_v):
    cid = lax.axis_index("c")
    sid = lax.axis_index("s")
    t = cid * NS + sid
    pltpu.sync_copy(col_hbm.at[pl.ds(t * NB, NB)], col_v)
    pltpu.sync_copy(ew_hbm.at[pl.ds(t * NB, NB)], ew_v)

    def _zero(i, _):
        deg_v[pl.ds(i * 16, 16)] = jnp.zeros((16,), jnp.float32)
        return 0

    lax.fori_loop(0, N // 16, _zero, 0)

    def _acc(i, _):
        b = i // (K // 16)
        g = i % (K // 16)
        idx = col_v[b, pl.ds(g * 16, 16)]
        val = ew_v[b, pl.ds(g * 16, 16)]
        plsc.addupdate_scatter(deg_v, [idx], val)
        return 0

    lax.fori_loop(0, NB * (K // 16), _acc, 0)
    pltpu.sync_copy(deg_v, degp_hbm.at[t, 0])


DH = D // 2

def _sc_agg_body(row_hbm, col_hbm, ew_hbm, ya_hbm, yb_hbm, outa_hbm, outb_hbm,
                 row_v, col_v, ew_v, rows0_v, rows1_v, zbuf_v, acc_s,
                 sg0, sg1, ss0, ss1):
    cid = lax.axis_index("c")
    sid = lax.axis_index("s")
    nb = jnp.where(cid == 0, NB0, NB1)
    start = pl.multiple_of(
        jnp.where(cid == 0, sid * NB0, NS * NB0 + sid * NB1), 8)

    pltpu.sync_copy(row_hbm.at[pl.ds(start, NBMX)], row_v.at[pl.ds(0, NBMX)])
    pltpu.sync_copy(col_hbm.at[pl.ds(start, NBMX)], col_v)
    pltpu.sync_copy(ew_hbm.at[pl.ds(start, NBMX)], ew_v)
    for d in range(2):
        for j in range(K // 16):
            row_v[nb + d, pl.ds(j * 16, 16)] = jnp.zeros((16,), jnp.int32)

    def _zrow(i, _):
        for j in range(DH // 32):
            zbuf_v[i, pl.ds(j * 32, 32)] = jnp.zeros((32,), jnp.bfloat16)
        return 0

    lax.fori_loop(0, K, _zrow, 0)
    base = pl.multiple_of(jnp.minimum(sid * RSTRIDE, N - RSPAN), 8)

    def _scale(rows_v, b):
        def _sg(g, _):
            ew16 = ew_v[b, pl.ds(g * 16, 16)]
            for l in range(16):
                sv = jnp.full((16,), ew16[l], jnp.float32)
                sb = plsc.pack(sv, sv, format=plsc.PackFormat.INTERLEAVED)
                e = g * 16 + l
                for j in range(DH // 32):
                    sl = rows_v[e, pl.ds(j * 32, 32)]
                    rows_v[e, pl.ds(j * 32, 32)] = sl * sb
            return 0

        lax.fori_loop(0, K // 16, _sg, 0)

    for y_hbm, out_hbm in ((ya_hbm, outa_hbm), (yb_hbm, outb_hbm)):
        for c in range(RSPAN // K):
            pltpu.sync_copy(zbuf_v, acc_s.at[pl.ds(base + c * K, K)])
        plsc.subcore_barrier()

        pltpu.async_copy(y_hbm.at[row_v.at[0]], rows0_v, sg0)
        pltpu.async_copy(y_hbm.at[row_v.at[1]], rows1_v, sg1)

        def _batch2(i, _):
            b = i * 2
            pltpu.make_async_copy(y_hbm.at[row_v.at[b]], rows0_v, sg0).wait()
            _scale(rows0_v, b)
            cp0 = pltpu.async_copy(rows0_v, acc_s.at[col_v.at[b]], ss0,
                                   add=True)
            pltpu.make_async_copy(y_hbm.at[row_v.at[b + 1]], rows1_v,
                                  sg1).wait()
            _scale(rows1_v, b + 1)
            cp1 = pltpu.async_copy(rows1_v, acc_s.at[col_v.at[b + 1]], ss1,
                                   add=True)
            cp0.wait()
            pltpu.async_copy(y_hbm.at[row_v.at[b + 2]], rows0_v, sg0)
            cp1.wait()
            pltpu.async_copy(y_hbm.at[row_v.at[b + 3]], rows1_v, sg1)
            return 0

        lax.fori_loop(0, nb // 2, _batch2, 0)
        pltpu.make_async_copy(y_hbm.at[row_v.at[nb]], rows0_v, sg0).wait()
        pltpu.make_async_copy(y_hbm.at[row_v.at[nb + 1]], rows1_v, sg1).wait()
        plsc.subcore_barrier()

        for c in range(RSPAN // K):
            pltpu.sync_copy(acc_s.at[pl.ds(base + c * K, K)], rows0_v)
            pltpu.sync_copy(rows0_v, out_hbm.at[cid, pl.ds(base + c * K, K)])
        plsc.subcore_barrier()


_sc_agg = pl.kernel(
    _sc_agg_body,
    out_type=(jax.ShapeDtypeStruct((NC, N, DH), jnp.bfloat16),
              jax.ShapeDtypeStruct((NC, N, DH), jnp.bfloat16)),
    mesh=_mesh,
    scratch_types=[
        pltpu.VMEM((NBMX + 2, K), jnp.int32),
        pltpu.VMEM((NBMX, K), jnp.int32),
        pltpu.VMEM((NBMX, K), jnp.float32),
        pltpu.VMEM((K, DH), jnp.bfloat16),
        pltpu.VMEM((K, DH), jnp.bfloat16),
        pltpu.VMEM((K, DH), jnp.bfloat16),
        pltpu.VMEM_SHARED((N, DH), jnp.bfloat16),
        pltpu.SemaphoreType.DMA,
        pltpu.SemaphoreType.DMA,
        pltpu.SemaphoreType.DMA,
        pltpu.SemaphoreType.DMA,
    ],
    compiler_params=_sc_params,
)


R = 1000
GRID = N // R


def _tc_pre_body(degp_ref, x_ref, w1_ref,
                 y1a_ref, y1b_ref, xw1_ref, dinv_ref, dinv2_ref):
    deg = 1.0 + jnp.sum(degp_ref[...], axis=0)
    dinv = lax.rsqrt(deg)
    dinv2 = dinv * dinv
    xw = jnp.dot(x_ref[...], w1_ref[...], preferred_element_type=jnp.float32)
    y1 = (dinv * xw).astype(jnp.bfloat16)
    xw1_ref[...] = xw
    y1a_ref[...] = y1[:, :DH]
    y1b_ref[...] = y1[:, DH:]
    dinv_ref[...] = dinv
    dinv2_ref[...] = dinv2


_tc_pre = pl.pallas_call(
    _tc_pre_body,
    grid=(GRID,),
    in_specs=[
        pl.BlockSpec((NW, R, 1), lambda i: (0, i, 0)),
        pl.BlockSpec((R, D), lambda i: (i, 0)),
        pl.BlockSpec((D, D), lambda i: (0, 0)),
    ],
    out_specs=[
        pl.BlockSpec((R, DH), lambda i: (i, 0)),
        pl.BlockSpec((R, DH), lambda i: (i, 0)),
        pl.BlockSpec((R, D), lambda i: (i, 0)),
        pl.BlockSpec((R, 1), lambda i: (i, 0)),
        pl.BlockSpec((R, 1), lambda i: (i, 0)),
    ],
    out_shape=[
        jax.ShapeDtypeStruct((N, DH), jnp.bfloat16),
        jax.ShapeDtypeStruct((N, DH), jnp.bfloat16),
        jax.ShapeDtypeStruct((N, D), jnp.float32),
        jax.ShapeDtypeStruct((N, 1), jnp.float32),
        jax.ShapeDtypeStruct((N, 1), jnp.float32),
    ],
)


def _tc_mid_body(s1pa_ref, s1pb_ref, xw1_ref, dinv_ref, dinv2_ref, b1_ref,
                 w2_ref, xw2_ref, y2a_ref, y2b_ref):
    f32 = jnp.float32
    s1 = jnp.concatenate(
        [s1pa_ref[0].astype(f32) + s1pa_ref[1].astype(f32),
         s1pb_ref[0].astype(f32) + s1pb_ref[1].astype(f32)], axis=1)
    dinv = dinv_ref[...]
    pre = dinv * s1 + dinv2_ref[...] * xw1_ref[...] + b1_ref[...]
    h = jnp.where(pre > 0, pre, jnp.exp(pre) - 1.0)
    xw2 = jnp.dot(h, w2_ref[...], preferred_element_type=jnp.float32)
    y2 = (dinv * xw2).astype(jnp.bfloat16)
    xw2_ref[...] = xw2
    y2a_ref[...] = y2[:, :DH]
    y2b_ref[...] = y2[:, DH:]


_tc_mid = pl.pallas_call(
    _tc_mid_body,
    grid=(GRID,),
    in_specs=[
        pl.BlockSpec((NC, R, DH), lambda i: (0, i, 0)),
        pl.BlockSpec((NC, R, DH), lambda i: (0, i, 0)),
        pl.BlockSpec((R, D), lambda i: (i, 0)),
        pl.BlockSpec((R, 1), lambda i: (i, 0)),
        pl.BlockSpec((R, 1), lambda i: (i, 0)),
        pl.BlockSpec((1, D), lambda i: (0, 0)),
        pl.BlockSpec((D, D), lambda i: (0, 0)),
    ],
    out_specs=[
        pl.BlockSpec((R, D), lambda i: (i, 0)),
        pl.BlockSpec((R, DH), lambda i: (i, 0)),
        pl.BlockSpec((R, DH), lambda i: (i, 0)),
    ],
    out_shape=[
        jax.ShapeDtypeStruct((N, D), jnp.float32),
        jax.ShapeDtypeStruct((N, DH), jnp.bfloat16),
        jax.ShapeDtypeStruct((N, DH), jnp.bfloat16),
    ],
)


def _tc_post_body(s2pa_ref, s2pb_ref, xw2_ref, dinv_ref, dinv2_ref, b2_ref,
                  out_ref):
    f32 = jnp.float32
    s2 = jnp.concatenate(
        [s2pa_ref[0].astype(f32) + s2pa_ref[1].astype(f32),
         s2pb_ref[0].astype(f32) + s2pb_ref[1].astype(f32)], axis=1)
    out_ref[...] = (dinv_ref[...] * s2 + dinv2_ref[...] * xw2_ref[...]
                    + b2_ref[...])


_tc_post = pl.pallas_call(
    _tc_post_body,
    grid=(GRID,),
    in_specs=[
        pl.BlockSpec((NC, R, DH), lambda i: (0, i, 0)),
        pl.BlockSpec((NC, R, DH), lambda i: (0, i, 0)),
        pl.BlockSpec((R, D), lambda i: (i, 0)),
        pl.BlockSpec((R, 1), lambda i: (i, 0)),
        pl.BlockSpec((R, 1), lambda i: (i, 0)),
        pl.BlockSpec((1, D), lambda i: (0, 0)),
    ],
    out_specs=pl.BlockSpec((R, D), lambda i: (i, 0)),
    out_shape=jax.ShapeDtypeStruct((N, D), jnp.float32),
)


def kernel(x, edge_index, edge_attr, W1, b1, W2, b2):
    row = edge_index[0]
    col = edge_index[1]
    pad = E_PAD - E
    row_p = jnp.concatenate([row, jnp.zeros((pad,), jnp.int32)])
    col_p = jnp.concatenate([col, jnp.zeros((pad,), jnp.int32)])
    ew_p = jnp.concatenate([edge_attr, jnp.zeros((pad,), jnp.float32)])
    row3 = row_p.reshape(NBT, K)
    col3 = col_p.reshape(NBT, K)
    ew3 = ew_p.reshape(NBT, K)

    degp = _sc_deg(col3, ew3)
    y1a, y1b, xw1, dinv, dinv2 = _tc_pre(degp.reshape(NW, N, 1), x, W1)
    s1pa, s1pb = _sc_agg(row3, col3, ew3, y1a, y1b)
    xw2, y2a, y2b = _tc_mid(s1pa, s1pb, xw1, dinv, dinv2,
                            b1.reshape(1, D), W2)
    s2pa, s2pb = _sc_agg(row3, col3, ew3, y2a, y2b)
    out = _tc_post(s2pa, s2pb, xw2, dinv, dinv2, b2.reshape(1, D))
    return out
